# baseline (speedup 1.0000x reference)
import functools

import jax
import jax.numpy as jnp
from jax import lax
from jax.experimental import pallas as pl
from jax.experimental.pallas import tpu as pltpu
from jax.experimental.pallas import tpu_sc as plsc

_EMBED_DIM = 32

_info = plsc.get_sparse_core_info()
_NC, _NS = _info.num_cores, _info.num_subcores
_NW = _NC * _NS


def _make_gather(batch: int, hist: int, chunk: int):
    b_per_w = batch // _NW
    rows_per_w = b_per_w * hist
    assert rows_per_w % chunk == 0
    n_chunks = rows_per_w // chunk
    n_rows = batch * hist
    mesh = plsc.VectorSubcoreMesh(core_axis_name="c", subcore_axis_name="s")

    @functools.partial(
        pl.kernel,
        mesh=mesh,
        compiler_params=pltpu.CompilerParams(
            use_tc_tiling_on_sc=False, needs_layout_passes=False
        ),
        out_type=jax.ShapeDtypeStruct((n_rows, _EMBED_DIM), jnp.float32),
        scratch_types=[
            pltpu.VMEM((hist, b_per_w), jnp.int32),
            pltpu.VMEM((rows_per_w,), jnp.int32),
            pltpu.VMEM((2, chunk, _EMBED_DIM), jnp.float32),
            pltpu.SemaphoreType.DMA,
            pltpu.SemaphoreType.DMA,
            pltpu.SemaphoreType.DMA,
            pltpu.SemaphoreType.DMA,
        ],
    )
    def gather_kernel(table_hbm, xt_hbm, out_hbm, idx2d, idx_perm, rows_v,
                      g0, g1, s0, s1):
        wid = lax.axis_index("s") * _NC + lax.axis_index("c")
        base = wid * rows_per_w
        pltpu.sync_copy(xt_hbm.at[:, pl.ds(wid * b_per_w, b_per_w)], idx2d)

        lanes = lax.iota(jnp.int32, 16)

        def perm_body(j, carry):
            m = j * 16 + lanes
            h = m % hist
            bb = m // hist
            v = plsc.load_gather(idx2d, [h, bb])
            idx_perm[pl.ds(j * 16, 16)] = v
            return carry

        lax.fori_loop(0, rows_per_w // 16, perm_body, 0)

        gsem = (g0, g1)
        ssem = (s0, s1)
        gathers = [None, None]
        stores = [None, None]
        gathers[0] = pltpu.async_copy(
            table_hbm.at[idx_perm.at[pl.ds(0, chunk)]], rows_v.at[0], g0
        )
        for i in range(n_chunks):
            b = i % 2
            nb = (i + 1) % 2
            if i + 1 < n_chunks:
                if stores[nb] is not None:
                    stores[nb].wait()
                gathers[nb] = pltpu.async_copy(
                    table_hbm.at[idx_perm.at[pl.ds((i + 1) * chunk, chunk)]],
                    rows_v.at[nb],
                    gsem[nb],
                )
            gathers[b].wait()
            stores[b] = pltpu.async_copy(
                rows_v.at[b], out_hbm.at[pl.ds(base + i * chunk, chunk)], ssem[b]
            )
        stores[(n_chunks - 1) % 2].wait()
        if n_chunks >= 2:
            stores[(n_chunks - 2) % 2].wait()

    return gather_kernel


def kernel(x, table):
    batch, hist = x.shape
    xt = x.T.astype(jnp.int32)
    out = _make_gather(batch, hist, 1280)(table, xt)
    return out.reshape(batch, hist, _EMBED_DIM)
